# baseline (speedup 1.0000x reference)
import functools

import jax
import jax.numpy as jnp
from jax import lax
from jax.experimental import pallas as pl
from jax.experimental.pallas import tpu as pltpu
from jax.experimental.pallas import tpu_sc as plsc

H = 128
N_PAD = 10240
SPLIT = 5120
EB = 4096
NSTEP = 20
E_PAD = 2 * NSTEP * EB


def _matmul_body(x_ref, ws_ref, wi_ref, wx_ref, ps_ref, pi_ref, px_ref):
    xb = x_ref[...]
    ps_ref[...] = jnp.dot(xb, ws_ref[...], preferred_element_type=jnp.float32)
    pi_ref[...] = jnp.dot(xb, wi_ref[...], preferred_element_type=jnp.float32)
    wxr = wx_ref[:, :H] + wx_ref[:, H:]
    px_ref[...] = jnp.dot(xb, wxr, preferred_element_type=jnp.float32)


def _matmuls(x_pad, W_sum, W_int, W_x):
    bm = 1024
    return pl.pallas_call(
        _matmul_body,
        grid=(N_PAD // bm,),
        in_specs=[
            pl.BlockSpec((bm, H), lambda i: (i, 0)),
            pl.BlockSpec((H, H), lambda i: (0, 0)),
            pl.BlockSpec((H, H), lambda i: (0, 0)),
            pl.BlockSpec((H, 2 * H), lambda i: (0, 0)),
        ],
        out_specs=[pl.BlockSpec((bm, H), lambda i: (i, 0))] * 3,
        out_shape=[jax.ShapeDtypeStruct((N_PAD, H), jnp.float32)] * 3,
    )(x_pad, W_sum, W_int, W_x)


def _ys_body(psum_hbm, dom_hbm, rb_hbm, z_hbm, ys_hbm,
             accum, zbuf, rbuf, ibuf, sbuf, rbv, dbuf, sem):
    c = lax.axis_index("c")
    s = lax.axis_index("s")
    iota = lax.iota(jnp.int32, 16)
    pltpu.sync_copy(z_hbm, zbuf)
    pltpu.sync_copy(rb_hbm.at[pl.ds(pl.multiple_of(8 * c, 8), 16)], rbv)
    base_z = 320 * s
    pltpu.sync_copy(zbuf, accum.at[pl.ds(base_z, 128)])
    pltpu.sync_copy(zbuf, accum.at[pl.ds(base_z + 128, 128)])
    pltpu.sync_copy(zbuf.at[pl.ds(0, 64)], accum.at[pl.ds(base_z + 256, 64)])

    @pl.when(s == 15)
    def _():
        pltpu.sync_copy(zbuf.at[pl.ds(0, 8)], accum.at[pl.ds(5120, 8)])

    plsc.subcore_barrier()

    rvec = rbv[...]
    b0 = rvec[0]
    b1 = rvec[1]
    j0 = b0 + ((b1 - b0) * s >> 4)
    j1 = b0 + ((b1 - b0) * (s + 1) >> 4)
    ja0 = j0 - (j0 & 7)
    trips = (j1 - ja0 + 127) >> 7
    id_base = SPLIT * c

    def ebody(t, carry):
        ja = pl.multiple_of(ja0 + (t << 7), 8)
        pltpu.sync_copy(dom_hbm.at[pl.ds(ja, 128)], ibuf)
        pltpu.sync_copy(psum_hbm.at[pl.ds(ja, 128)], rbuf)
        for v in range(8):
            sl = pl.ds(v * 16, 16)
            jv = ja + v * 16 + iota
            valid = (jv >= j0) & (jv < j1)
            sbuf[sl] = jnp.where(valid, ibuf[sl] - id_base, 5120)
        pltpu.async_copy(rbuf, accum.at[sbuf], sem, add=True).wait()
        return carry

    lax.fori_loop(0, trips, ebody, 0)
    plsc.subcore_barrier()

    for k in range(5):
        pltpu.sync_copy(accum.at[pl.ds(base_z + 64 * k, 64)], dbuf)
        pltpu.sync_copy(dbuf, ys_hbm.at[pl.ds(id_base + base_z + 64 * k, 64)])


def _ys_call(P_sum, dom_pad, rb, zrows):
    mesh = plsc.VectorSubcoreMesh(core_axis_name="c", subcore_axis_name="s")
    f = pl.kernel(
        _ys_body,
        out_type=jax.ShapeDtypeStruct((N_PAD, H), jnp.float32),
        mesh=mesh,
        scratch_types=[
            pltpu.VMEM_SHARED((5128, H), jnp.float32),
            pltpu.VMEM((128, H), jnp.float32),
            pltpu.VMEM((128, H), jnp.float32),
            pltpu.VMEM((128,), jnp.int32),
            pltpu.VMEM((128,), jnp.int32),
            pltpu.VMEM((16,), jnp.int32),
            pltpu.VMEM((64, H), jnp.float32),
            pltpu.SemaphoreType.DMA,
        ],
    )
    return f(P_sum, dom_pad, rb, zrows)


def _main_body(pint_hbm, px_hbm, ys_hbm, ii_hbm, nm_hbm, dm0_hbm, dm1_hbm,
               bnd_hbm, z_hbm, msg_hbm,
               accum, zbuf, rowsA, rowsB, rowsC, rowsD, iislab, nmslab,
               gi0, gi1, gi2, gi3, si0, si1, si2, si3, oi0, oi1,
               bndv, sem_i, sem_g, sem_s):
    c = lax.axis_index("c")
    s = lax.axis_index("s")
    iota = lax.iota(jnp.int32, 16)
    rowsb = [rowsA, rowsB, rowsC, rowsD]
    gis = [gi0, gi1, gi2, gi3]
    sis = [si0, si1, si2, si3]
    ois = [oi0, oi1]
    pltpu.sync_copy(z_hbm, zbuf)
    for m in range(2):
        for v in range(8):
            ois[m][pl.ds(v * 16, 16)] = 256 * s + 128 * m + v * 16 + iota

    def step_body(t, carry):
        b = c * NSTEP + t
        e_lo = b << 12
        pltpu.sync_copy(zbuf, accum.at[pl.ds(256 * s, 128)])
        pltpu.sync_copy(zbuf, accum.at[pl.ds(256 * s + 128, 128)])
        plsc.subcore_barrier()
        pltpu.sync_copy(bnd_hbm.at[pl.ds(pl.multiple_of(b << 3, 8), 16)], bndv)
        bvec = bndv[...]
        b0 = bvec[0]
        b1 = bvec[1]
        j0 = b0 + ((b1 - b0) * s >> 4)
        j1 = b0 + ((b1 - b0) * (s + 1) >> 4)
        ja0 = j0 - (j0 & 7)
        trips = (j1 - ja0 + 511) >> 9

        def ebody(u, ecarry):
            ja = pl.multiple_of(ja0 + (u << 9), 8)
            pltpu.sync_copy(ii_hbm.at[pl.ds(ja, 512)], iislab)
            pltpu.sync_copy(nm_hbm.at[pl.ds(ja, 512)], nmslab)
            for v in range(32):
                k, sl = v >> 3, pl.ds(v * 16, 16)
                ksl = pl.ds((v & 7) * 16, 16)
                jv = ja + v * 16 + iota
                valid = (jv >= j0) & (jv < j1)
                gis[k][ksl] = jnp.where(valid, nmslab[sl], 10000)
                sis[k][ksl] = jnp.where(valid, iislab[sl] - e_lo, 0)
            for k in range(4):
                pltpu.async_copy(pint_hbm.at[gis[k]], rowsb[k], sem_g).wait()
                pltpu.async_copy(rowsb[k], accum.at[sis[k]], sem_s,
                                 add=True).wait()
            return ecarry

        lax.fori_loop(0, trips, ebody, 0)
        plsc.subcore_barrier()
        a0 = pl.multiple_of(e_lo + 256 * s, 8)
        a1 = pl.multiple_of(e_lo + 256 * s + 128, 8)
        pltpu.sync_copy(dm1_hbm.at[pl.ds(a0, 128)], gi0)
        pltpu.sync_copy(dm0_hbm.at[pl.ds(a0, 128)], gi1)
        pltpu.sync_copy(dm1_hbm.at[pl.ds(a1, 128)], gi2)
        pltpu.sync_copy(dm0_hbm.at[pl.ds(a1, 128)], gi3)
        pltpu.async_copy(ys_hbm.at[gi0], rowsA, sem_g).wait()
        pltpu.async_copy(rowsA, accum.at[ois[0]], sem_s, add=True).wait()
        pltpu.async_copy(px_hbm.at[gi1], rowsB, sem_g).wait()
        pltpu.async_copy(rowsB, accum.at[ois[0]], sem_s, add=True).wait()
        pltpu.async_copy(ys_hbm.at[gi2], rowsC, sem_g).wait()
        pltpu.async_copy(rowsC, accum.at[ois[1]], sem_s, add=True).wait()
        pltpu.async_copy(px_hbm.at[gi3], rowsD, sem_g).wait()
        pltpu.async_copy(rowsD, accum.at[ois[1]], sem_s, add=True).wait()
        pltpu.sync_copy(accum.at[pl.ds(256 * s, 128)], rowsA)
        pltpu.sync_copy(rowsA, msg_hbm.at[pl.ds(a0, 128)])
        pltpu.sync_copy(accum.at[pl.ds(256 * s + 128, 128)], rowsB)
        pltpu.sync_copy(rowsB, msg_hbm.at[pl.ds(a1, 128)])
        plsc.subcore_barrier()
        return carry

    lax.fori_loop(0, NSTEP, step_body, 0)


def _main_call(P_int, P_x, ys, ii_pad, nm_pad, dm0, dm1, bnd, zrows):
    mesh = plsc.VectorSubcoreMesh(core_axis_name="c", subcore_axis_name="s")
    f = pl.kernel(
        _main_body,
        out_type=jax.ShapeDtypeStruct((E_PAD, H), jnp.float32),
        mesh=mesh,
        scratch_types=(
            [pltpu.VMEM_SHARED((EB, H), jnp.float32)]
            + [pltpu.VMEM((128, H), jnp.float32)] * 5
            + [pltpu.VMEM((512,), jnp.int32),
               pltpu.VMEM((512,), jnp.int32)]
            + [pltpu.VMEM((128,), jnp.int32)] * 10
            + [pltpu.VMEM((16,), jnp.int32),
               pltpu.SemaphoreType.DMA,
               pltpu.SemaphoreType.DMA,
               pltpu.SemaphoreType.DMA]
        ),
    )
    return f(P_int, P_x, ys, ii_pad, nm_pad, dm0, dm1, bnd, zrows)


def _bn_body(ed_f, msg_ref, g_ref, bta_ref, out_ref, s_acc, q_acc):
    p = pl.program_id(0)
    i = pl.program_id(1)

    @pl.when(jnp.logical_and(p == 0, i == 0))
    def _():
        s_acc[...] = jnp.zeros_like(s_acc)
        q_acc[...] = jnp.zeros_like(q_acc)

    blk = msg_ref[...]

    @pl.when(p == 0)
    def _():
        s_acc[...] += jnp.sum(blk, axis=0, keepdims=True)
        q_acc[...] += jnp.sum(blk * blk, axis=0, keepdims=True)
        out_ref[...] = blk

    @pl.when(p == 1)
    def _():
        mean = s_acc[...] / ed_f
        var = q_acc[...] / ed_f - mean * mean
        inv = lax.rsqrt(var + 1e-5) * g_ref[...]
        out_ref[...] = jnp.maximum((blk - mean) * inv + bta_ref[...], 0.0)


def _bn_call(msg, gamma, beta, ed):
    bm = 2048
    return pl.pallas_call(
        functools.partial(_bn_body, float(ed)),
        grid=(2, E_PAD // bm),
        in_specs=[
            pl.BlockSpec((bm, H), lambda p, i: (i, 0)),
            pl.BlockSpec((1, H), lambda p, i: (0, 0)),
            pl.BlockSpec((1, H), lambda p, i: (0, 0)),
        ],
        out_specs=pl.BlockSpec((bm, H), lambda p, i: (i, 0)),
        out_shape=jax.ShapeDtypeStruct((E_PAD, H), jnp.float32),
        scratch_shapes=[
            pltpu.VMEM((1, H), jnp.float32),
            pltpu.VMEM((1, H), jnp.float32),
        ],
    )(msg, gamma, beta)


def kernel(x, y, domain_indicator, node_map_edge_index, intersect_indicator,
           domain_map_edge_index, W_sum, W_int, W_x, bn_gamma, bn_beta):
    n = x.shape[0]
    en = node_map_edge_index.shape[1]
    ed = domain_map_edge_index.shape[1]

    x_pad = jnp.zeros((N_PAD, H), jnp.float32).at[:n].set(x)
    P_sum, P_int, P_x = _matmuls(x_pad, W_sum, W_int, W_x)

    dom = domain_indicator.astype(jnp.int32)
    dom_pad = jnp.zeros((N_PAD,), jnp.int32).at[:n].set(dom)
    rb1 = jnp.searchsorted(dom, SPLIT).astype(jnp.int32)
    rb = (jnp.zeros((24,), jnp.int32)
          .at[1].set(rb1).at[8].set(rb1).at[9].set(n))
    zrows = jnp.zeros((128, H), jnp.float32)
    ys = _ys_call(P_sum, dom_pad, rb, zrows)

    ii = intersect_indicator.astype(jnp.int32)
    ii_pad = jnp.zeros((en + 512,), jnp.int32).at[:en].set(ii)
    nm_pad = jnp.zeros((en + 512,), jnp.int32).at[:en].set(
        node_map_edge_index[1].astype(jnp.int32))
    nblocks = E_PAD // EB
    bounds = jnp.searchsorted(
        ii, jnp.arange(nblocks + 1, dtype=jnp.int32) * EB).astype(jnp.int32)
    ar = jnp.arange(nblocks) * 8
    bnd = (jnp.zeros((8 * nblocks + 16,), jnp.int32)
           .at[ar].set(bounds[:nblocks])
           .at[ar + 1].set(bounds[1:nblocks + 1]))
    dm0 = jnp.full((E_PAD,), n, jnp.int32).at[:ed].set(
        domain_map_edge_index[0].astype(jnp.int32))
    dm1 = jnp.full((E_PAD,), n, jnp.int32).at[:ed].set(
        domain_map_edge_index[1].astype(jnp.int32))

    msg = _main_call(P_int, P_x, ys, ii_pad, nm_pad, dm0, dm1, bnd, zrows)
    out = _bn_call(msg, bn_gamma.reshape(1, H), bn_beta.reshape(1, H), ed)
    return out[:ed]


# speedup vs baseline: 3.1983x; 3.1983x over previous
import functools

import jax
import jax.numpy as jnp
from jax import lax
from jax.experimental import pallas as pl
from jax.experimental.pallas import tpu as pltpu
from jax.experimental.pallas import tpu_sc as plsc

H = 128
N_PAD = 10240
SPLIT = 5120
EB = 4096
NSTEP = 20
E_PAD = 2 * NSTEP * EB


def _matmul_body(x_ref, ws_ref, wi_ref, wx_ref, ps_ref, pi_ref, px_ref):
    xb = x_ref[...]
    ps_ref[...] = jnp.dot(xb, ws_ref[...], preferred_element_type=jnp.float32)
    pi_ref[...] = jnp.dot(xb, wi_ref[...], preferred_element_type=jnp.float32)
    wxr = wx_ref[:, :H] + wx_ref[:, H:]
    px_ref[...] = jnp.dot(xb, wxr, preferred_element_type=jnp.float32)


def _matmuls(x_pad, W_sum, W_int, W_x):
    bm = 1024
    return pl.pallas_call(
        _matmul_body,
        grid=(N_PAD // bm,),
        in_specs=[
            pl.BlockSpec((bm, H), lambda i: (i, 0)),
            pl.BlockSpec((H, H), lambda i: (0, 0)),
            pl.BlockSpec((H, H), lambda i: (0, 0)),
            pl.BlockSpec((H, 2 * H), lambda i: (0, 0)),
        ],
        out_specs=[pl.BlockSpec((bm, H), lambda i: (i, 0))] * 3,
        out_shape=[jax.ShapeDtypeStruct((N_PAD, H), jnp.float32)] * 3,
    )(x_pad, W_sum, W_int, W_x)


def _ys_body(psum_hbm, dom_hbm, rb_hbm, z_hbm, ys_hbm,
             accum, zbuf, rbuf, ibuf, sbuf, rbv, dbuf, sem):
    c = lax.axis_index("c")
    s = lax.axis_index("s")
    iota = lax.iota(jnp.int32, 16)
    pltpu.sync_copy(z_hbm, zbuf)
    pltpu.sync_copy(rb_hbm.at[pl.ds(pl.multiple_of(8 * c, 8), 16)], rbv)
    base_z = 320 * s
    pltpu.sync_copy(zbuf, accum.at[pl.ds(base_z, 128)])
    pltpu.sync_copy(zbuf, accum.at[pl.ds(base_z + 128, 128)])
    pltpu.sync_copy(zbuf.at[pl.ds(0, 64)], accum.at[pl.ds(base_z + 256, 64)])

    @pl.when(s == 15)
    def _():
        pltpu.sync_copy(zbuf.at[pl.ds(0, 8)], accum.at[pl.ds(5120, 8)])

    plsc.subcore_barrier()

    rvec = rbv[...]
    b0 = rvec[0]
    b1 = rvec[1]
    j0 = b0 + ((b1 - b0) * s >> 4)
    j1 = b0 + ((b1 - b0) * (s + 1) >> 4)
    ja0 = j0 - (j0 & 7)
    trips = (j1 - ja0 + 127) >> 7
    id_base = SPLIT * c

    def ebody(t, carry):
        ja = pl.multiple_of(ja0 + (t << 7), 8)
        pltpu.sync_copy(dom_hbm.at[pl.ds(ja, 128)], ibuf)
        pltpu.sync_copy(psum_hbm.at[pl.ds(ja, 128)], rbuf)
        for v in range(8):
            sl = pl.ds(v * 16, 16)
            jv = ja + v * 16 + iota
            valid = (jv >= j0) & (jv < j1)
            sbuf[sl] = jnp.where(valid, ibuf[sl] - id_base, 5120)
        pltpu.async_copy(rbuf, accum.at[sbuf], sem, add=True).wait()
        return carry

    lax.fori_loop(0, trips, ebody, 0)
    plsc.subcore_barrier()

    for k in range(5):
        pltpu.sync_copy(accum.at[pl.ds(base_z + 64 * k, 64)], dbuf)
        pltpu.sync_copy(dbuf, ys_hbm.at[pl.ds(id_base + base_z + 64 * k, 64)])


def _ys_call(P_sum, dom_pad, rb, zrows):
    mesh = plsc.VectorSubcoreMesh(core_axis_name="c", subcore_axis_name="s")
    f = pl.kernel(
        _ys_body,
        out_type=jax.ShapeDtypeStruct((N_PAD, H), jnp.float32),
        mesh=mesh,
        scratch_types=[
            pltpu.VMEM_SHARED((5128, H), jnp.float32),
            pltpu.VMEM((128, H), jnp.float32),
            pltpu.VMEM((128, H), jnp.float32),
            pltpu.VMEM((128,), jnp.int32),
            pltpu.VMEM((128,), jnp.int32),
            pltpu.VMEM((16,), jnp.int32),
            pltpu.VMEM((64, H), jnp.float32),
            pltpu.SemaphoreType.DMA,
        ],
    )
    return f(P_sum, dom_pad, rb, zrows)


def _main_body(pint_hbm, px_hbm, ys_hbm, ii_hbm, nm_hbm, dm0_hbm, dm1_hbm,
               bnd_hbm, z_hbm, msg_hbm,
               accum, zbuf, rowsA, gi0, gi1, si0, oi0, bndv, sem_g, sem_s):
    c = lax.axis_index("c")
    s = lax.axis_index("s")
    iota = lax.iota(jnp.int32, 16)
    pltpu.sync_copy(z_hbm, zbuf)

    def step_body(t, carry):
        b = c * NSTEP + t
        e_lo = b << 12
        def zbody(m, zc):
            pltpu.sync_copy(
                zbuf, accum.at[pl.ds(pl.multiple_of(256 * s + 128 * m, 8), 128)])
            return zc

        lax.fori_loop(0, 2, zbody, 0)
        plsc.subcore_barrier()
        pltpu.sync_copy(bnd_hbm.at[pl.ds(pl.multiple_of(b << 3, 8), 16)], bndv)
        bvec = bndv[...]
        b0 = bvec[0]
        b1 = bvec[1]
        j0 = b0 + ((b1 - b0) * s >> 4)
        j1 = b0 + ((b1 - b0) * (s + 1) >> 4)
        ja0 = j0 - (j0 & 7)
        trips = (j1 - ja0 + 127) >> 7

        def ebody(u, ecarry):
            ja = pl.multiple_of(ja0 + (u << 7), 8)
            pltpu.sync_copy(ii_hbm.at[pl.ds(ja, 128)], gi1)
            pltpu.sync_copy(nm_hbm.at[pl.ds(ja, 128)], si0)
            for v in range(8):
                sl = pl.ds(v * 16, 16)
                jv = ja + v * 16 + iota
                valid = (jv >= j0) & (jv < j1)
                gi0[sl] = jnp.where(valid, si0[sl], 10000)
                si0[sl] = jnp.where(valid, gi1[sl] - e_lo, 0)
            pltpu.async_copy(pint_hbm.at[gi0], rowsA, sem_g).wait()
            pltpu.async_copy(rowsA, accum.at[si0], sem_s, add=True).wait()
            return ecarry

        lax.fori_loop(0, trips, ebody, 0)
        plsc.subcore_barrier()

        def obody(k, oc):
            la = 256 * s + 128 * k
            a = pl.multiple_of(e_lo + la, 8)
            for v in range(8):
                oi0[pl.ds(v * 16, 16)] = la + v * 16 + iota
            pltpu.sync_copy(dm1_hbm.at[pl.ds(a, 128)], gi1)
            pltpu.async_copy(ys_hbm.at[gi1], rowsA, sem_g).wait()
            pltpu.async_copy(rowsA, accum.at[oi0], sem_s, add=True).wait()
            pltpu.sync_copy(dm0_hbm.at[pl.ds(a, 128)], gi1)
            pltpu.async_copy(px_hbm.at[gi1], rowsA, sem_g).wait()
            pltpu.async_copy(rowsA, accum.at[oi0], sem_s, add=True).wait()
            pltpu.sync_copy(
                accum.at[pl.ds(pl.multiple_of(la, 8), 128)], rowsA)
            pltpu.sync_copy(rowsA, msg_hbm.at[pl.ds(a, 128)])
            return oc

        lax.fori_loop(0, 2, obody, 0)
        plsc.subcore_barrier()
        return carry

    lax.fori_loop(0, NSTEP, step_body, 0)


def _main_call(P_int, P_x, ys, ii_pad, nm_pad, dm0, dm1, bnd, zrows):
    mesh = plsc.VectorSubcoreMesh(core_axis_name="c", subcore_axis_name="s")
    f = pl.kernel(
        _main_body,
        out_type=jax.ShapeDtypeStruct((E_PAD, H), jnp.float32),
        mesh=mesh,
        scratch_types=(
            [pltpu.VMEM_SHARED((EB, H), jnp.float32),
             pltpu.VMEM((128, H), jnp.float32),
             pltpu.VMEM((128, H), jnp.float32)]
            + [pltpu.VMEM((128,), jnp.int32)] * 4
            + [pltpu.VMEM((16,), jnp.int32),
               pltpu.SemaphoreType.DMA,
               pltpu.SemaphoreType.DMA]
        ),
    )
    return f(P_int, P_x, ys, ii_pad, nm_pad, dm0, dm1, bnd, zrows)


def _bn_body(ed_f, msg_ref, g_ref, bta_ref, out_ref, s_acc, q_acc):
    p = pl.program_id(0)
    i = pl.program_id(1)

    @pl.when(jnp.logical_and(p == 0, i == 0))
    def _():
        s_acc[...] = jnp.zeros_like(s_acc)
        q_acc[...] = jnp.zeros_like(q_acc)

    blk = msg_ref[...]

    @pl.when(p == 0)
    def _():
        s_acc[...] += jnp.sum(blk, axis=0, keepdims=True)
        q_acc[...] += jnp.sum(blk * blk, axis=0, keepdims=True)
        out_ref[...] = blk

    @pl.when(p == 1)
    def _():
        mean = s_acc[...] / ed_f
        var = q_acc[...] / ed_f - mean * mean
        inv = lax.rsqrt(var + 1e-5) * g_ref[...]
        out_ref[...] = jnp.maximum((blk - mean) * inv + bta_ref[...], 0.0)


def _bn_call(msg, gamma, beta, ed):
    bm = 2048
    return pl.pallas_call(
        functools.partial(_bn_body, float(ed)),
        grid=(2, E_PAD // bm),
        in_specs=[
            pl.BlockSpec((bm, H), lambda p, i: (i, 0)),
            pl.BlockSpec((1, H), lambda p, i: (0, 0)),
            pl.BlockSpec((1, H), lambda p, i: (0, 0)),
        ],
        out_specs=pl.BlockSpec((bm, H), lambda p, i: (i, 0)),
        out_shape=jax.ShapeDtypeStruct((E_PAD, H), jnp.float32),
        scratch_shapes=[
            pltpu.VMEM((1, H), jnp.float32),
            pltpu.VMEM((1, H), jnp.float32),
        ],
    )(msg, gamma, beta)


def kernel(x, y, domain_indicator, node_map_edge_index, intersect_indicator,
           domain_map_edge_index, W_sum, W_int, W_x, bn_gamma, bn_beta):
    n = x.shape[0]
    en = node_map_edge_index.shape[1]
    ed = domain_map_edge_index.shape[1]

    x_pad = jnp.zeros((N_PAD, H), jnp.float32).at[:n].set(x)
    P_sum, P_int, P_x = _matmuls(x_pad, W_sum, W_int, W_x)

    dom = domain_indicator.astype(jnp.int32)
    dom_pad = jnp.zeros((N_PAD,), jnp.int32).at[:n].set(dom)
    rb1 = jnp.searchsorted(dom, SPLIT).astype(jnp.int32)
    rb = (jnp.zeros((24,), jnp.int32)
          .at[1].set(rb1).at[8].set(rb1).at[9].set(n))
    zrows = jnp.zeros((128, H), jnp.float32)
    ys = _ys_call(P_sum, dom_pad, rb, zrows)

    ii = intersect_indicator.astype(jnp.int32)
    ii_pad = jnp.zeros((en + 512,), jnp.int32).at[:en].set(ii)
    nm_pad = jnp.zeros((en + 512,), jnp.int32).at[:en].set(
        node_map_edge_index[1].astype(jnp.int32))
    nblocks = E_PAD // EB
    bounds = jnp.searchsorted(
        ii, jnp.arange(nblocks + 1, dtype=jnp.int32) * EB).astype(jnp.int32)
    ar = jnp.arange(nblocks) * 8
    bnd = (jnp.zeros((8 * nblocks + 16,), jnp.int32)
           .at[ar].set(bounds[:nblocks])
           .at[ar + 1].set(bounds[1:nblocks + 1]))
    dm0 = jnp.full((E_PAD,), n, jnp.int32).at[:ed].set(
        domain_map_edge_index[0].astype(jnp.int32))
    dm1 = jnp.full((E_PAD,), n, jnp.int32).at[:ed].set(
        domain_map_edge_index[1].astype(jnp.int32))

    msg = _main_call(P_int, P_x, ys, ii_pad, nm_pad, dm0, dm1, bnd, zrows)
    out = _bn_call(msg, bn_gamma.reshape(1, H), bn_beta.reshape(1, H), ed)
    return out[:ed]


# speedup vs baseline: 5.5508x; 1.7355x over previous
import functools

import jax
import jax.numpy as jnp
from jax import lax
from jax.experimental import pallas as pl
from jax.experimental.pallas import tpu as pltpu
from jax.experimental.pallas import tpu_sc as plsc

H = 128
N_PAD = 10240
SPLIT = 5120
EB = 8192
NSTEP = 10
E_PAD = 2 * NSTEP * EB


def _matmul_body(x_ref, ws_ref, wi_ref, wx_ref, ps_ref, pi_ref, px_ref):
    xb = x_ref[...]
    ps_ref[...] = jnp.dot(xb, ws_ref[...], preferred_element_type=jnp.float32)
    pi_ref[...] = jnp.dot(xb, wi_ref[...], preferred_element_type=jnp.float32)
    wxr = wx_ref[:, :H] + wx_ref[:, H:]
    px_ref[...] = jnp.dot(xb, wxr, preferred_element_type=jnp.float32)


def _matmuls(x_pad, W_sum, W_int, W_x):
    bm = 1024
    return pl.pallas_call(
        _matmul_body,
        grid=(N_PAD // bm,),
        in_specs=[
            pl.BlockSpec((bm, H), lambda i: (i, 0)),
            pl.BlockSpec((H, H), lambda i: (0, 0)),
            pl.BlockSpec((H, H), lambda i: (0, 0)),
            pl.BlockSpec((H, 2 * H), lambda i: (0, 0)),
        ],
        out_specs=[pl.BlockSpec((bm, H), lambda i: (i, 0))] * 3,
        out_shape=[jax.ShapeDtypeStruct((N_PAD, H), jnp.float32)] * 3,
    )(x_pad, W_sum, W_int, W_x)


def _ys_body(psum_hbm, dom_hbm, rb_hbm, z_hbm, ys_hbm,
             accum, zbuf, rbuf, ibuf, sbuf, rbv, dbuf, sem):
    c = lax.axis_index("c")
    s = lax.axis_index("s")
    iota = lax.iota(jnp.int32, 16)
    pltpu.sync_copy(z_hbm, zbuf)
    pltpu.sync_copy(rb_hbm.at[pl.ds(pl.multiple_of(8 * c, 8), 16)], rbv)
    base_z = 320 * s
    pltpu.sync_copy(zbuf, accum.at[pl.ds(base_z, 128)])
    pltpu.sync_copy(zbuf, accum.at[pl.ds(base_z + 128, 128)])
    pltpu.sync_copy(zbuf.at[pl.ds(0, 64)], accum.at[pl.ds(base_z + 256, 64)])

    @pl.when(s == 15)
    def _():
        pltpu.sync_copy(zbuf.at[pl.ds(0, 8)], accum.at[pl.ds(5120, 8)])

    plsc.subcore_barrier()

    rvec = rbv[...]
    b0 = rvec[0]
    b1 = rvec[1]
    j0 = b0 + ((b1 - b0) * s >> 4)
    j1 = b0 + ((b1 - b0) * (s + 1) >> 4)
    ja0 = j0 - (j0 & 7)
    trips = (j1 - ja0 + 127) >> 7
    id_base = SPLIT * c

    def ebody(t, carry):
        ja = pl.multiple_of(ja0 + (t << 7), 8)
        pltpu.sync_copy(dom_hbm.at[pl.ds(ja, 128)], ibuf)
        pltpu.sync_copy(psum_hbm.at[pl.ds(ja, 128)], rbuf)
        for v in range(8):
            sl = pl.ds(v * 16, 16)
            jv = ja + v * 16 + iota
            valid = (jv >= j0) & (jv < j1)
            sbuf[sl] = jnp.where(valid, ibuf[sl] - id_base, 5120)
        pltpu.async_copy(rbuf, accum.at[sbuf], sem, add=True).wait()
        return carry

    lax.fori_loop(0, trips, ebody, 0)
    plsc.subcore_barrier()

    for k in range(5):
        pltpu.sync_copy(accum.at[pl.ds(base_z + 64 * k, 64)], dbuf)
        pltpu.sync_copy(dbuf, ys_hbm.at[pl.ds(id_base + base_z + 64 * k, 64)])


def _ys_call(P_sum, dom_pad, rb, zrows):
    mesh = plsc.VectorSubcoreMesh(core_axis_name="c", subcore_axis_name="s")
    f = pl.kernel(
        _ys_body,
        out_type=jax.ShapeDtypeStruct((N_PAD, H), jnp.float32),
        mesh=mesh,
        scratch_types=[
            pltpu.VMEM_SHARED((5128, H), jnp.float32),
            pltpu.VMEM((128, H), jnp.float32),
            pltpu.VMEM((128, H), jnp.float32),
            pltpu.VMEM((128,), jnp.int32),
            pltpu.VMEM((128,), jnp.int32),
            pltpu.VMEM((16,), jnp.int32),
            pltpu.VMEM((64, H), jnp.float32),
            pltpu.SemaphoreType.DMA,
        ],
    )
    return f(P_sum, dom_pad, rb, zrows)


def _main_body(pint_hbm, px_hbm, ys_hbm, ii_hbm, nm_hbm, dm0_hbm, dm1_hbm,
               bnd_hbm, z_hbm, msg_hbm,
               accum, zbuf, rowsA, rowsB, gi1, giA, giB, siA, siB, oi0, bndv,
               semA, semB, sem_s):
    c = lax.axis_index("c")
    s = lax.axis_index("s")
    iota = lax.iota(jnp.int32, 16)
    pltpu.sync_copy(z_hbm, zbuf)
    rowsb = [rowsA, rowsB]
    gib = [giA, giB]
    sib = [siA, siB]
    semb = [semA, semB]

    def step_body(t, carry):
        b = c * NSTEP + t
        e_lo = b << 13
        def zbody(m, zc):
            pltpu.sync_copy(
                zbuf, accum.at[pl.ds(pl.multiple_of(512 * s + 128 * m, 8), 128)])
            return zc

        lax.fori_loop(0, 4, zbody, 0)
        plsc.subcore_barrier()
        pltpu.sync_copy(bnd_hbm.at[pl.ds(pl.multiple_of(b << 3, 8), 16)], bndv)
        bvec = bndv[...]
        b0 = bvec[0]
        b1 = bvec[1]
        j0 = b0 + ((b1 - b0) * s >> 4)
        j1 = b0 + ((b1 - b0) * (s + 1) >> 4)
        ja0 = j0 - (j0 & 7)
        trips = (j1 - ja0 + 127) >> 7

        def fire_g(pp, u):
            ja = pl.multiple_of(ja0 + (u << 7), 8)
            pltpu.sync_copy(ii_hbm.at[pl.ds(ja, 128)], gi1)
            pltpu.sync_copy(nm_hbm.at[pl.ds(ja, 128)], sib[pp])
            for v in range(8):
                sl = pl.ds(v * 16, 16)
                jv = ja + v * 16 + iota
                valid = (jv >= j0) & (jv < j1)
                gib[pp][sl] = jnp.where(valid, sib[pp][sl], 10000)
                sib[pp][sl] = jnp.where(valid, gi1[sl] - e_lo, 0)
            pltpu.async_copy(pint_hbm.at[gib[pp]], rowsb[pp], semb[pp])

        def wait_gs(pp):
            pltpu.make_async_copy(pint_hbm.at[gib[pp]], rowsb[pp],
                                  semb[pp]).wait()
            pltpu.async_copy(rowsb[pp], accum.at[sib[pp]], sem_s,
                             add=True).wait()

        def ebody(u, ecarry):
            p = u & 1

            @pl.when((u < trips) & (p == 0))
            def _():
                fire_g(0, u)

            @pl.when((u < trips) & (p == 1))
            def _():
                fire_g(1, u)

            @pl.when((u >= 1) & (p == 1))
            def _():
                wait_gs(0)

            @pl.when((u >= 1) & (p == 0))
            def _():
                wait_gs(1)

            return ecarry

        lax.fori_loop(0, trips + 1, ebody, 0)
        plsc.subcore_barrier()

        def obody(k, oc):
            la = 512 * s + 128 * k
            a = pl.multiple_of(e_lo + la, 8)
            for v in range(8):
                oi0[pl.ds(v * 16, 16)] = la + v * 16 + iota
            pltpu.sync_copy(dm1_hbm.at[pl.ds(a, 128)], giA)
            pltpu.sync_copy(dm0_hbm.at[pl.ds(a, 128)], giB)
            dg = [pltpu.async_copy(ys_hbm.at[giA], rowsA, semA),
                  pltpu.async_copy(px_hbm.at[giB], rowsB, semB)]
            for d in dg:
                d.wait()
            ds_ = [pltpu.async_copy(rowsA, accum.at[oi0], sem_s, add=True),
                   pltpu.async_copy(rowsB, accum.at[oi0], sem_s, add=True)]
            for d in ds_:
                d.wait()
            pltpu.sync_copy(
                accum.at[pl.ds(pl.multiple_of(la, 8), 128)], rowsA)
            pltpu.sync_copy(rowsA, msg_hbm.at[pl.ds(a, 128)])
            return oc

        lax.fori_loop(0, 4, obody, 0)
        plsc.subcore_barrier()
        return carry

    lax.fori_loop(0, NSTEP, step_body, 0)


def _main_call(P_int, P_x, ys, ii_pad, nm_pad, dm0, dm1, bnd, zrows):
    mesh = plsc.VectorSubcoreMesh(core_axis_name="c", subcore_axis_name="s")
    f = pl.kernel(
        _main_body,
        out_type=jax.ShapeDtypeStruct((E_PAD, H), jnp.float32),
        mesh=mesh,
        scratch_types=(
            [pltpu.VMEM_SHARED((EB, H), jnp.float32)]
            + [pltpu.VMEM((128, H), jnp.float32)] * 3
            + [pltpu.VMEM((128,), jnp.int32)] * 6
            + [pltpu.VMEM((16,), jnp.int32),
               pltpu.SemaphoreType.DMA,
               pltpu.SemaphoreType.DMA,
               pltpu.SemaphoreType.DMA]
        ),
    )
    return f(P_int, P_x, ys, ii_pad, nm_pad, dm0, dm1, bnd, zrows)


def _bn_body(ed_f, msg_ref, g_ref, bta_ref, out_ref, s_acc, q_acc):
    p = pl.program_id(0)
    i = pl.program_id(1)

    @pl.when(jnp.logical_and(p == 0, i == 0))
    def _():
        s_acc[...] = jnp.zeros_like(s_acc)
        q_acc[...] = jnp.zeros_like(q_acc)

    blk = msg_ref[...]

    @pl.when(p == 0)
    def _():
        s_acc[...] += jnp.sum(blk, axis=0, keepdims=True)
        q_acc[...] += jnp.sum(blk * blk, axis=0, keepdims=True)
        out_ref[...] = blk

    @pl.when(p == 1)
    def _():
        mean = s_acc[...] / ed_f
        var = q_acc[...] / ed_f - mean * mean
        inv = lax.rsqrt(var + 1e-5) * g_ref[...]
        out_ref[...] = jnp.maximum((blk - mean) * inv + bta_ref[...], 0.0)


def _bn_call(msg, gamma, beta, ed):
    bm = 2048
    return pl.pallas_call(
        functools.partial(_bn_body, float(ed)),
        grid=(2, E_PAD // bm),
        in_specs=[
            pl.BlockSpec((bm, H), lambda p, i: (i, 0)),
            pl.BlockSpec((1, H), lambda p, i: (0, 0)),
            pl.BlockSpec((1, H), lambda p, i: (0, 0)),
        ],
        out_specs=pl.BlockSpec((bm, H), lambda p, i: (i, 0)),
        out_shape=jax.ShapeDtypeStruct((E_PAD, H), jnp.float32),
        scratch_shapes=[
            pltpu.VMEM((1, H), jnp.float32),
            pltpu.VMEM((1, H), jnp.float32),
        ],
    )(msg, gamma, beta)


def kernel(x, y, domain_indicator, node_map_edge_index, intersect_indicator,
           domain_map_edge_index, W_sum, W_int, W_x, bn_gamma, bn_beta):
    n = x.shape[0]
    en = node_map_edge_index.shape[1]
    ed = domain_map_edge_index.shape[1]

    x_pad = jnp.zeros((N_PAD, H), jnp.float32).at[:n].set(x)
    P_sum, P_int, P_x = _matmuls(x_pad, W_sum, W_int, W_x)

    dom = domain_indicator.astype(jnp.int32)
    dom_pad = jnp.zeros((N_PAD,), jnp.int32).at[:n].set(dom)
    rb1 = jnp.searchsorted(dom, SPLIT).astype(jnp.int32)
    rb = (jnp.zeros((24,), jnp.int32)
          .at[1].set(rb1).at[8].set(rb1).at[9].set(n))
    zrows = jnp.zeros((128, H), jnp.float32)
    ys = _ys_call(P_sum, dom_pad, rb, zrows)

    ii = intersect_indicator.astype(jnp.int32)
    ii_pad = jnp.zeros((en + 512,), jnp.int32).at[:en].set(ii)
    nm_pad = jnp.zeros((en + 512,), jnp.int32).at[:en].set(
        node_map_edge_index[1].astype(jnp.int32))
    nblocks = E_PAD // EB
    bounds = jnp.searchsorted(
        ii, jnp.arange(nblocks + 1, dtype=jnp.int32) * EB).astype(jnp.int32)
    ar = jnp.arange(nblocks) * 8
    bnd = (jnp.zeros((8 * nblocks + 16,), jnp.int32)
           .at[ar].set(bounds[:nblocks])
           .at[ar + 1].set(bounds[1:nblocks + 1]))
    dm0 = jnp.full((E_PAD,), n, jnp.int32).at[:ed].set(
        domain_map_edge_index[0].astype(jnp.int32))
    dm1 = jnp.full((E_PAD,), n, jnp.int32).at[:ed].set(
        domain_map_edge_index[1].astype(jnp.int32))

    msg = _main_call(P_int, P_x, ys, ii_pad, nm_pad, dm0, dm1, bnd, zrows)
    out = _bn_call(msg, bn_gamma.reshape(1, H), bn_beta.reshape(1, H), ed)
    return out[:ed]


# speedup vs baseline: 5.7333x; 1.0329x over previous
import functools

import jax
import jax.numpy as jnp
from jax import lax
from jax.experimental import pallas as pl
from jax.experimental.pallas import tpu as pltpu
from jax.experimental.pallas import tpu_sc as plsc

H = 128
N_PAD = 10240
SPLIT = 5120
EB = 8192
NSTEP = 10
E_PAD = 2 * NSTEP * EB


def _matmul_body(x_ref, ws_ref, wi_ref, wx_ref, ps_ref, pi_ref, px_ref):
    xb = x_ref[...]
    ps_ref[...] = jnp.dot(xb, ws_ref[...], preferred_element_type=jnp.float32)
    pi_ref[...] = jnp.dot(xb, wi_ref[...], preferred_element_type=jnp.float32)
    wxr = wx_ref[:, :H] + wx_ref[:, H:]
    px_ref[...] = jnp.dot(xb, wxr, preferred_element_type=jnp.float32)


def _matmuls(x_pad, W_sum, W_int, W_x):
    bm = 1024
    return pl.pallas_call(
        _matmul_body,
        grid=(N_PAD // bm,),
        in_specs=[
            pl.BlockSpec((bm, H), lambda i: (i, 0)),
            pl.BlockSpec((H, H), lambda i: (0, 0)),
            pl.BlockSpec((H, H), lambda i: (0, 0)),
            pl.BlockSpec((H, 2 * H), lambda i: (0, 0)),
        ],
        out_specs=[pl.BlockSpec((bm, H), lambda i: (i, 0))] * 3,
        out_shape=[jax.ShapeDtypeStruct((N_PAD, H), jnp.float32)] * 3,
    )(x_pad, W_sum, W_int, W_x)


def _ys_body(psum_hbm, dom_hbm, rb_hbm, z_hbm, ys_hbm,
             accum, zbuf, rbuf, ibuf, sbuf, rbv, dbuf, sem):
    c = lax.axis_index("c")
    s = lax.axis_index("s")
    iota = lax.iota(jnp.int32, 16)
    pltpu.sync_copy(z_hbm, zbuf)
    pltpu.sync_copy(rb_hbm.at[pl.ds(pl.multiple_of(8 * c, 8), 16)], rbv)
    base_z = 320 * s
    pltpu.sync_copy(zbuf, accum.at[pl.ds(base_z, 128)])
    pltpu.sync_copy(zbuf, accum.at[pl.ds(base_z + 128, 128)])
    pltpu.sync_copy(zbuf.at[pl.ds(0, 64)], accum.at[pl.ds(base_z + 256, 64)])

    @pl.when(s == 15)
    def _():
        pltpu.sync_copy(zbuf.at[pl.ds(0, 8)], accum.at[pl.ds(5120, 8)])

    plsc.subcore_barrier()

    rvec = rbv[...]
    b0 = rvec[0]
    b1 = rvec[1]
    j0 = b0 + ((b1 - b0) * s >> 4)
    j1 = b0 + ((b1 - b0) * (s + 1) >> 4)
    ja0 = j0 - (j0 & 7)
    trips = (j1 - ja0 + 127) >> 7
    id_base = SPLIT * c

    def ebody(t, carry):
        ja = pl.multiple_of(ja0 + (t << 7), 8)
        pltpu.sync_copy(dom_hbm.at[pl.ds(ja, 128)], ibuf)
        pltpu.sync_copy(psum_hbm.at[pl.ds(ja, 128)], rbuf)
        for v in range(8):
            sl = pl.ds(v * 16, 16)
            jv = ja + v * 16 + iota
            valid = (jv >= j0) & (jv < j1)
            sbuf[sl] = jnp.where(valid, ibuf[sl] - id_base, 5120)
        pltpu.async_copy(rbuf, accum.at[sbuf], sem, add=True).wait()
        return carry

    lax.fori_loop(0, trips, ebody, 0)
    plsc.subcore_barrier()

    for k in range(5):
        pltpu.sync_copy(accum.at[pl.ds(base_z + 64 * k, 64)], dbuf)
        pltpu.sync_copy(dbuf, ys_hbm.at[pl.ds(id_base + base_z + 64 * k, 64)])


def _ys_call(P_sum, dom_pad, rb, zrows):
    mesh = plsc.VectorSubcoreMesh(core_axis_name="c", subcore_axis_name="s")
    f = pl.kernel(
        _ys_body,
        out_type=jax.ShapeDtypeStruct((N_PAD, H), jnp.float32),
        mesh=mesh,
        scratch_types=[
            pltpu.VMEM_SHARED((5128, H), jnp.float32),
            pltpu.VMEM((128, H), jnp.float32),
            pltpu.VMEM((128, H), jnp.float32),
            pltpu.VMEM((128,), jnp.int32),
            pltpu.VMEM((128,), jnp.int32),
            pltpu.VMEM((16,), jnp.int32),
            pltpu.VMEM((64, H), jnp.float32),
            pltpu.SemaphoreType.DMA,
        ],
    )
    return f(P_sum, dom_pad, rb, zrows)


def _main_body(pint_hbm, px_hbm, ys_hbm, ii_hbm, nm_hbm, dm0_hbm, dm1_hbm,
               bnd_hbm, z_hbm, msg_hbm,
               accum, zbuf, rowsA, rowsB, iiA, iiB, nmA, nmB, giA, giB,
               siA, siB, oi0,
               bndv, semA, semB, sem_s, semIA, semIB, semZ):
    c = lax.axis_index("c")
    s = lax.axis_index("s")
    iota = lax.iota(jnp.int32, 16)
    pltpu.sync_copy(z_hbm, zbuf)
    rowsb = [rowsA, rowsB]
    iib = [iiA, iiB]
    nmb = [nmA, nmB]
    gib = [giA, giB]
    sib = [siA, siB]
    semb = [semA, semB]
    semi = [semIA, semIB]

    def step_body(t, carry):
        b = c * NSTEP + t
        e_lo = b << 13
        def zfire(m, zc):
            pltpu.async_copy(
                zbuf, accum.at[pl.ds(pl.multiple_of(512 * s + 128 * m, 8), 128)],
                semZ)
            return zc

        def zwait(m, zc):
            pltpu.make_async_copy(
                zbuf, accum.at[pl.ds(pl.multiple_of(512 * s + 128 * m, 8), 128)],
                semZ).wait()
            return zc

        lax.fori_loop(0, 4, zfire, 0)
        lax.fori_loop(0, 4, zwait, 0)
        plsc.subcore_barrier()
        pltpu.sync_copy(bnd_hbm.at[pl.ds(pl.multiple_of(b << 3, 8), 16)], bndv)
        bvec = bndv[...]
        b0 = bvec[0]
        b1 = bvec[1]
        j0 = b0 + ((b1 - b0) * s >> 4)
        j1 = b0 + ((b1 - b0) * (s + 1) >> 4)
        ja0 = j0 - (j0 & 7)
        trips = (j1 - ja0 + 127) >> 7

        def chunk_ja(u):
            return pl.multiple_of(ja0 + (u << 7), 8)

        def fire_idx(pp, u):
            ja = chunk_ja(u)
            pltpu.async_copy(ii_hbm.at[pl.ds(ja, 128)], iib[pp], semi[pp])
            pltpu.async_copy(nm_hbm.at[pl.ds(ja, 128)], nmb[pp], semi[pp])

        def fire_g(pp, u):
            ja = chunk_ja(u)
            pltpu.make_async_copy(ii_hbm.at[pl.ds(ja, 128)], iib[pp],
                                  semi[pp]).wait()
            pltpu.make_async_copy(nm_hbm.at[pl.ds(ja, 128)], nmb[pp],
                                  semi[pp]).wait()
            for v in range(8):
                sl = pl.ds(v * 16, 16)
                jv = ja + v * 16 + iota
                valid = (jv >= j0) & (jv < j1)
                gib[pp][sl] = jnp.where(valid, nmb[pp][sl], 10000)
                sib[pp][sl] = jnp.where(valid, iib[pp][sl] - e_lo, 0)
            pltpu.async_copy(pint_hbm.at[gib[pp]], rowsb[pp], semb[pp])

        def wait_gs(pp):
            pltpu.make_async_copy(pint_hbm.at[gib[pp]], rowsb[pp],
                                  semb[pp]).wait()
            pltpu.async_copy(rowsb[pp], accum.at[sib[pp]], sem_s,
                             add=True).wait()

        @pl.when(trips >= 1)
        def _():
            fire_idx(0, 0)

        def ebody(u, ecarry):
            p = u & 1

            @pl.when((u < trips) & (p == 0))
            def _():
                fire_g(0, u)

            @pl.when((u < trips) & (p == 1))
            def _():
                fire_g(1, u)

            @pl.when((u + 1 < trips) & (p == 0))
            def _():
                fire_idx(1, u + 1)

            @pl.when((u + 1 < trips) & (p == 1))
            def _():
                fire_idx(0, u + 1)

            @pl.when((u >= 1) & (p == 1))
            def _():
                wait_gs(0)

            @pl.when((u >= 1) & (p == 0))
            def _():
                wait_gs(1)

            return ecarry

        lax.fori_loop(0, trips + 1, ebody, 0)
        plsc.subcore_barrier()

        def obody(k, oc):
            la = 512 * s + 128 * k
            a = pl.multiple_of(e_lo + la, 8)
            for v in range(8):
                oi0[pl.ds(v * 16, 16)] = la + v * 16 + iota
            di = [pltpu.async_copy(dm1_hbm.at[pl.ds(a, 128)], giA, semIA),
                  pltpu.async_copy(dm0_hbm.at[pl.ds(a, 128)], giB, semIB)]
            for d in di:
                d.wait()
            dg = [pltpu.async_copy(ys_hbm.at[giA], rowsA, semA),
                  pltpu.async_copy(px_hbm.at[giB], rowsB, semB)]
            for d in dg:
                d.wait()
            ds_ = [pltpu.async_copy(rowsA, accum.at[oi0], sem_s, add=True),
                   pltpu.async_copy(rowsB, accum.at[oi0], sem_s, add=True)]
            for d in ds_:
                d.wait()
            pltpu.sync_copy(
                accum.at[pl.ds(pl.multiple_of(la, 8), 128)], rowsA)
            pltpu.sync_copy(rowsA, msg_hbm.at[pl.ds(a, 128)])
            return oc

        lax.fori_loop(0, 4, obody, 0)
        plsc.subcore_barrier()
        return carry

    lax.fori_loop(0, NSTEP, step_body, 0)


def _main_call(P_int, P_x, ys, ii_pad, nm_pad, dm0, dm1, bnd, zrows):
    mesh = plsc.VectorSubcoreMesh(core_axis_name="c", subcore_axis_name="s")
    f = pl.kernel(
        _main_body,
        out_type=jax.ShapeDtypeStruct((E_PAD, H), jnp.float32),
        mesh=mesh,
        scratch_types=(
            [pltpu.VMEM_SHARED((EB, H), jnp.float32)]
            + [pltpu.VMEM((128, H), jnp.float32)] * 3
            + [pltpu.VMEM((128,), jnp.int32)] * 9
            + [pltpu.VMEM((16,), jnp.int32)]
            + [pltpu.SemaphoreType.DMA] * 6
        ),
    )
    return f(P_int, P_x, ys, ii_pad, nm_pad, dm0, dm1, bnd, zrows)


def _bn_body(ed_f, msg_ref, g_ref, bta_ref, out_ref, s_acc, q_acc):
    p = pl.program_id(0)
    i = pl.program_id(1)

    @pl.when(jnp.logical_and(p == 0, i == 0))
    def _():
        s_acc[...] = jnp.zeros_like(s_acc)
        q_acc[...] = jnp.zeros_like(q_acc)

    blk = msg_ref[...]

    @pl.when(p == 0)
    def _():
        s_acc[...] += jnp.sum(blk, axis=0, keepdims=True)
        q_acc[...] += jnp.sum(blk * blk, axis=0, keepdims=True)
        out_ref[...] = blk

    @pl.when(p == 1)
    def _():
        mean = s_acc[...] / ed_f
        var = q_acc[...] / ed_f - mean * mean
        inv = lax.rsqrt(var + 1e-5) * g_ref[...]
        out_ref[...] = jnp.maximum((blk - mean) * inv + bta_ref[...], 0.0)


def _bn_call(msg, gamma, beta, ed):
    bm = 2048
    return pl.pallas_call(
        functools.partial(_bn_body, float(ed)),
        grid=(2, E_PAD // bm),
        in_specs=[
            pl.BlockSpec((bm, H), lambda p, i: (i, 0)),
            pl.BlockSpec((1, H), lambda p, i: (0, 0)),
            pl.BlockSpec((1, H), lambda p, i: (0, 0)),
        ],
        out_specs=pl.BlockSpec((bm, H), lambda p, i: (i, 0)),
        out_shape=jax.ShapeDtypeStruct((E_PAD, H), jnp.float32),
        scratch_shapes=[
            pltpu.VMEM((1, H), jnp.float32),
            pltpu.VMEM((1, H), jnp.float32),
        ],
    )(msg, gamma, beta)


def kernel(x, y, domain_indicator, node_map_edge_index, intersect_indicator,
           domain_map_edge_index, W_sum, W_int, W_x, bn_gamma, bn_beta):
    n = x.shape[0]
    en = node_map_edge_index.shape[1]
    ed = domain_map_edge_index.shape[1]

    x_pad = jnp.zeros((N_PAD, H), jnp.float32).at[:n].set(x)
    P_sum, P_int, P_x = _matmuls(x_pad, W_sum, W_int, W_x)

    dom = domain_indicator.astype(jnp.int32)
    dom_pad = jnp.zeros((N_PAD,), jnp.int32).at[:n].set(dom)
    rb1 = jnp.searchsorted(dom, SPLIT).astype(jnp.int32)
    rb = (jnp.zeros((24,), jnp.int32)
          .at[1].set(rb1).at[8].set(rb1).at[9].set(n))
    zrows = jnp.zeros((128, H), jnp.float32)
    ys = _ys_call(P_sum, dom_pad, rb, zrows)

    ii = intersect_indicator.astype(jnp.int32)
    ii_pad = jnp.zeros((en + 512,), jnp.int32).at[:en].set(ii)
    nm_pad = jnp.zeros((en + 512,), jnp.int32).at[:en].set(
        node_map_edge_index[1].astype(jnp.int32))
    nblocks = E_PAD // EB
    bounds = jnp.searchsorted(
        ii, jnp.arange(nblocks + 1, dtype=jnp.int32) * EB).astype(jnp.int32)
    ar = jnp.arange(nblocks) * 8
    bnd = (jnp.zeros((8 * nblocks + 16,), jnp.int32)
           .at[ar].set(bounds[:nblocks])
           .at[ar + 1].set(bounds[1:nblocks + 1]))
    dm0 = jnp.full((E_PAD,), n, jnp.int32).at[:ed].set(
        domain_map_edge_index[0].astype(jnp.int32))
    dm1 = jnp.full((E_PAD,), n, jnp.int32).at[:ed].set(
        domain_map_edge_index[1].astype(jnp.int32))

    msg = _main_call(P_int, P_x, ys, ii_pad, nm_pad, dm0, dm1, bnd, zrows)
    out = _bn_call(msg, bn_gamma.reshape(1, H), bn_beta.reshape(1, H), ed)
    return out[:ed]


# speedup vs baseline: 5.9869x; 1.0442x over previous
import functools

import jax
import jax.numpy as jnp
from jax import lax
from jax.experimental import pallas as pl
from jax.experimental.pallas import tpu as pltpu
from jax.experimental.pallas import tpu_sc as plsc

H = 128
N_PAD = 10240
SPLIT = 5120
EB = 8192
NSTEP = 10
E_PAD = 2 * NSTEP * EB


def _matmul_body(x_ref, ws_ref, wi_ref, wx_ref, ps_ref, pi_ref, px_ref):
    xb = x_ref[...]
    ps_ref[...] = jnp.dot(xb, ws_ref[...], preferred_element_type=jnp.float32)
    pi_ref[...] = jnp.dot(xb, wi_ref[...], preferred_element_type=jnp.float32)
    wxr = wx_ref[:, :H] + wx_ref[:, H:]
    px_ref[...] = jnp.dot(xb, wxr, preferred_element_type=jnp.float32)


def _matmuls(x_pad, W_sum, W_int, W_x):
    bm = 1024
    return pl.pallas_call(
        _matmul_body,
        grid=(N_PAD // bm,),
        in_specs=[
            pl.BlockSpec((bm, H), lambda i: (i, 0)),
            pl.BlockSpec((H, H), lambda i: (0, 0)),
            pl.BlockSpec((H, H), lambda i: (0, 0)),
            pl.BlockSpec((H, 2 * H), lambda i: (0, 0)),
        ],
        out_specs=[pl.BlockSpec((bm, H), lambda i: (i, 0))] * 3,
        out_shape=[jax.ShapeDtypeStruct((N_PAD, H), jnp.float32)] * 3,
    )(x_pad, W_sum, W_int, W_x)


def _ys_body(psum_hbm, dom_hbm, rb_hbm, z_hbm, ys_hbm,
             accum, zbuf, rbuf, ibuf, sbuf, rbv, dbuf, sem):
    c = lax.axis_index("c")
    s = lax.axis_index("s")
    iota = lax.iota(jnp.int32, 16)
    pltpu.sync_copy(z_hbm, zbuf)
    pltpu.sync_copy(rb_hbm.at[pl.ds(pl.multiple_of(8 * c, 8), 16)], rbv)
    base_z = 320 * s
    pltpu.sync_copy(zbuf, accum.at[pl.ds(base_z, 128)])
    pltpu.sync_copy(zbuf, accum.at[pl.ds(base_z + 128, 128)])
    pltpu.sync_copy(zbuf.at[pl.ds(0, 64)], accum.at[pl.ds(base_z + 256, 64)])

    @pl.when(s == 15)
    def _():
        pltpu.sync_copy(zbuf.at[pl.ds(0, 8)], accum.at[pl.ds(5120, 8)])

    plsc.subcore_barrier()

    rvec = rbv[...]
    b0 = rvec[0]
    b1 = rvec[1]
    j0 = b0 + ((b1 - b0) * s >> 4)
    j1 = b0 + ((b1 - b0) * (s + 1) >> 4)
    ja0 = j0 - (j0 & 7)
    trips = (j1 - ja0 + 127) >> 7
    id_base = SPLIT * c

    def ebody(t, carry):
        ja = pl.multiple_of(ja0 + (t << 7), 8)
        pltpu.sync_copy(dom_hbm.at[pl.ds(ja, 128)], ibuf)
        pltpu.sync_copy(psum_hbm.at[pl.ds(ja, 128)], rbuf)
        for v in range(8):
            sl = pl.ds(v * 16, 16)
            jv = ja + v * 16 + iota
            valid = (jv >= j0) & (jv < j1)
            sbuf[sl] = jnp.where(valid, ibuf[sl] - id_base, 5120)
        pltpu.async_copy(rbuf, accum.at[sbuf], sem, add=True).wait()
        return carry

    lax.fori_loop(0, trips, ebody, 0)
    plsc.subcore_barrier()

    for k in range(5):
        pltpu.sync_copy(accum.at[pl.ds(base_z + 64 * k, 64)], dbuf)
        pltpu.sync_copy(dbuf, ys_hbm.at[pl.ds(id_base + base_z + 64 * k, 64)])


def _ys_call(P_sum, dom_pad, rb, zrows):
    mesh = plsc.VectorSubcoreMesh(core_axis_name="c", subcore_axis_name="s")
    f = pl.kernel(
        _ys_body,
        out_type=jax.ShapeDtypeStruct((N_PAD, H), jnp.float32),
        mesh=mesh,
        scratch_types=[
            pltpu.VMEM_SHARED((5128, H), jnp.float32),
            pltpu.VMEM((128, H), jnp.float32),
            pltpu.VMEM((128, H), jnp.float32),
            pltpu.VMEM((128,), jnp.int32),
            pltpu.VMEM((128,), jnp.int32),
            pltpu.VMEM((16,), jnp.int32),
            pltpu.VMEM((64, H), jnp.float32),
            pltpu.SemaphoreType.DMA,
        ],
    )
    return f(P_sum, dom_pad, rb, zrows)


def _main_body(pint_hbm, px_hbm, ys_hbm, ii_hbm, nm_hbm, dm0_hbm, dm1_hbm,
               bnd_hbm, z_hbm, msg_hbm,
               accum, zbuf, rowsA, rowsB, iiA, iiB, nmA, nmB, giA, giB,
               siA, siB, oi0,
               bndv, semA, semB, sem_s, semIA, semIB, semZ):
    c = lax.axis_index("c")
    s = lax.axis_index("s")
    iota = lax.iota(jnp.int32, 16)
    pltpu.sync_copy(z_hbm, zbuf)
    rowsb = [rowsA, rowsB]
    iib = [iiA, iiB]
    nmb = [nmA, nmB]
    gib = [giA, giB]
    sib = [siA, siB]
    semb = [semA, semB]
    semi = [semIA, semIB]

    def step_body(t, carry):
        b = c * NSTEP + t
        e_lo = b << 13
        def zfire(m, zc):
            pltpu.async_copy(
                zbuf, accum.at[pl.ds(pl.multiple_of(512 * s + 128 * m, 8), 128)],
                semZ)
            return zc

        def zwait(m, zc):
            pltpu.make_async_copy(
                zbuf, accum.at[pl.ds(pl.multiple_of(512 * s + 128 * m, 8), 128)],
                semZ).wait()
            return zc

        lax.fori_loop(0, 4, zfire, 0)
        lax.fori_loop(0, 4, zwait, 0)
        plsc.subcore_barrier()
        pltpu.sync_copy(bnd_hbm.at[pl.ds(pl.multiple_of(b << 3, 8), 16)], bndv)
        bvec = bndv[...]
        b0 = bvec[0]
        b1 = bvec[1]
        j0 = b0 + ((b1 - b0) * s >> 4)
        j1 = b0 + ((b1 - b0) * (s + 1) >> 4)
        ja0 = j0 - (j0 & 7)
        trips = (j1 - ja0 + 127) >> 7

        def chunk_ja(u):
            return pl.multiple_of(ja0 + (u << 7), 8)

        def fire_idx(pp, u):
            ja = chunk_ja(u)
            pltpu.async_copy(ii_hbm.at[pl.ds(ja, 128)], iib[pp], semi[pp])
            pltpu.async_copy(nm_hbm.at[pl.ds(ja, 128)], nmb[pp], semi[pp])

        def fire_g(pp, u):
            ja = chunk_ja(u)
            pltpu.make_async_copy(ii_hbm.at[pl.ds(ja, 128)], iib[pp],
                                  semi[pp]).wait()
            pltpu.make_async_copy(nm_hbm.at[pl.ds(ja, 128)], nmb[pp],
                                  semi[pp]).wait()
            for v in range(8):
                sl = pl.ds(v * 16, 16)
                jv = ja + v * 16 + iota
                valid = (jv >= j0) & (jv < j1)
                gib[pp][sl] = jnp.where(valid, nmb[pp][sl], 10000)
                sib[pp][sl] = jnp.where(valid, iib[pp][sl] - e_lo, 0)
            pltpu.async_copy(pint_hbm.at[gib[pp]], rowsb[pp], semb[pp])

        def wait_gs(pp):
            pltpu.make_async_copy(pint_hbm.at[gib[pp]], rowsb[pp],
                                  semb[pp]).wait()
            pltpu.async_copy(rowsb[pp], accum.at[sib[pp]], sem_s,
                             add=True).wait()

        @pl.when(trips >= 1)
        def _():
            fire_idx(0, 0)

        def ebody(u, ecarry):
            p = u & 1

            @pl.when((u < trips) & (p == 0))
            def _():
                fire_g(0, u)

            @pl.when((u < trips) & (p == 1))
            def _():
                fire_g(1, u)

            @pl.when((u + 1 < trips) & (p == 0))
            def _():
                fire_idx(1, u + 1)

            @pl.when((u + 1 < trips) & (p == 1))
            def _():
                fire_idx(0, u + 1)

            @pl.when((u >= 1) & (p == 1))
            def _():
                wait_gs(0)

            @pl.when((u >= 1) & (p == 0))
            def _():
                wait_gs(1)

            return ecarry

        lax.fori_loop(0, trips + 1, ebody, 0)
        plsc.subcore_barrier()

        def ofire(pp, k):
            a = pl.multiple_of(e_lo + 512 * s + 128 * k, 8)
            pltpu.async_copy(dm1_hbm.at[pl.ds(a, 128)], iib[pp], semi[pp])
            pltpu.async_copy(dm0_hbm.at[pl.ds(a, 128)], nmb[pp], semi[pp])

        def owork(pp, k):
            la = 512 * s + 128 * k
            a = pl.multiple_of(e_lo + la, 8)
            for v in range(8):
                oi0[pl.ds(v * 16, 16)] = la + v * 16 + iota
            pltpu.make_async_copy(dm1_hbm.at[pl.ds(a, 128)], iib[pp],
                                  semi[pp]).wait()
            pltpu.make_async_copy(dm0_hbm.at[pl.ds(a, 128)], nmb[pp],
                                  semi[pp]).wait()
            dg = [pltpu.async_copy(ys_hbm.at[iib[pp]], rowsA, semA),
                  pltpu.async_copy(px_hbm.at[nmb[pp]], rowsB, semB)]
            for d in dg:
                d.wait()
            ds_ = [pltpu.async_copy(rowsA, accum.at[oi0], sem_s, add=True),
                   pltpu.async_copy(rowsB, accum.at[oi0], sem_s, add=True)]
            for d in ds_:
                d.wait()
            pltpu.sync_copy(accum.at[pl.ds(pl.multiple_of(la, 8), 128)],
                            msg_hbm.at[pl.ds(a, 128)])

        ofire(0, 0)

        def obody(k, oc):
            p = k & 1

            @pl.when((k + 1 < 4) & (p == 0))
            def _():
                ofire(1, k + 1)

            @pl.when((k + 1 < 4) & (p == 1))
            def _():
                ofire(0, k + 1)

            @pl.when(p == 0)
            def _():
                owork(0, k)

            @pl.when(p == 1)
            def _():
                owork(1, k)

            return oc

        lax.fori_loop(0, 4, obody, 0)
        plsc.subcore_barrier()
        return carry

    lax.fori_loop(0, NSTEP, step_body, 0)


def _main_call(P_int, P_x, ys, ii_pad, nm_pad, dm0, dm1, bnd, zrows):
    mesh = plsc.VectorSubcoreMesh(core_axis_name="c", subcore_axis_name="s")
    f = pl.kernel(
        _main_body,
        out_type=jax.ShapeDtypeStruct((E_PAD, H), jnp.float32),
        mesh=mesh,
        scratch_types=(
            [pltpu.VMEM_SHARED((EB, H), jnp.float32)]
            + [pltpu.VMEM((128, H), jnp.float32)] * 3
            + [pltpu.VMEM((128,), jnp.int32)] * 9
            + [pltpu.VMEM((16,), jnp.int32)]
            + [pltpu.SemaphoreType.DMA] * 6
        ),
    )
    return f(P_int, P_x, ys, ii_pad, nm_pad, dm0, dm1, bnd, zrows)


def _bn_body(ed_f, msg_ref, g_ref, bta_ref, out_ref, s_acc, q_acc):
    p = pl.program_id(0)
    i = pl.program_id(1)

    @pl.when(jnp.logical_and(p == 0, i == 0))
    def _():
        s_acc[...] = jnp.zeros_like(s_acc)
        q_acc[...] = jnp.zeros_like(q_acc)

    blk = msg_ref[...]

    @pl.when(p == 0)
    def _():
        s_acc[...] += jnp.sum(blk, axis=0, keepdims=True)
        q_acc[...] += jnp.sum(blk * blk, axis=0, keepdims=True)
        out_ref[...] = blk

    @pl.when(p == 1)
    def _():
        mean = s_acc[...] / ed_f
        var = q_acc[...] / ed_f - mean * mean
        inv = lax.rsqrt(var + 1e-5) * g_ref[...]
        out_ref[...] = jnp.maximum((blk - mean) * inv + bta_ref[...], 0.0)


def _bn_call(msg, gamma, beta, ed):
    bm = 2048
    return pl.pallas_call(
        functools.partial(_bn_body, float(ed)),
        grid=(2, E_PAD // bm),
        in_specs=[
            pl.BlockSpec((bm, H), lambda p, i: (i, 0)),
            pl.BlockSpec((1, H), lambda p, i: (0, 0)),
            pl.BlockSpec((1, H), lambda p, i: (0, 0)),
        ],
        out_specs=pl.BlockSpec((bm, H), lambda p, i: (i, 0)),
        out_shape=jax.ShapeDtypeStruct((E_PAD, H), jnp.float32),
        scratch_shapes=[
            pltpu.VMEM((1, H), jnp.float32),
            pltpu.VMEM((1, H), jnp.float32),
        ],
    )(msg, gamma, beta)


def kernel(x, y, domain_indicator, node_map_edge_index, intersect_indicator,
           domain_map_edge_index, W_sum, W_int, W_x, bn_gamma, bn_beta):
    n = x.shape[0]
    en = node_map_edge_index.shape[1]
    ed = domain_map_edge_index.shape[1]

    x_pad = jnp.zeros((N_PAD, H), jnp.float32).at[:n].set(x)
    P_sum, P_int, P_x = _matmuls(x_pad, W_sum, W_int, W_x)

    dom = domain_indicator.astype(jnp.int32)
    dom_pad = jnp.zeros((N_PAD,), jnp.int32).at[:n].set(dom)
    rb1 = jnp.searchsorted(dom, SPLIT).astype(jnp.int32)
    rb = (jnp.zeros((24,), jnp.int32)
          .at[1].set(rb1).at[8].set(rb1).at[9].set(n))
    zrows = jnp.zeros((128, H), jnp.float32)
    ys = _ys_call(P_sum, dom_pad, rb, zrows)

    ii = intersect_indicator.astype(jnp.int32)
    ii_pad = jnp.zeros((en + 512,), jnp.int32).at[:en].set(ii)
    nm_pad = jnp.zeros((en + 512,), jnp.int32).at[:en].set(
        node_map_edge_index[1].astype(jnp.int32))
    nblocks = E_PAD // EB
    bounds = jnp.searchsorted(
        ii, jnp.arange(nblocks + 1, dtype=jnp.int32) * EB).astype(jnp.int32)
    ar = jnp.arange(nblocks) * 8
    bnd = (jnp.zeros((8 * nblocks + 16,), jnp.int32)
           .at[ar].set(bounds[:nblocks])
           .at[ar + 1].set(bounds[1:nblocks + 1]))
    dm0 = jnp.full((E_PAD,), n, jnp.int32).at[:ed].set(
        domain_map_edge_index[0].astype(jnp.int32))
    dm1 = jnp.full((E_PAD,), n, jnp.int32).at[:ed].set(
        domain_map_edge_index[1].astype(jnp.int32))

    msg = _main_call(P_int, P_x, ys, ii_pad, nm_pad, dm0, dm1, bnd, zrows)
    out = _bn_call(msg, bn_gamma.reshape(1, H), bn_beta.reshape(1, H), ed)
    return out[:ed]
